# baseline (speedup 1.0000x reference)
import functools

import jax
import jax.numpy as jnp
from jax import lax
from jax.experimental import pallas as pl
from jax.experimental.pallas import tpu as pltpu
from jax.experimental.pallas import tpu_sc as plsc

D = 1024
E = 8
F = 4096
S = 2048
T = 128
NT = (2 * S + E * (T - 1) + T - 1) // T
P = NT * T
FT = 512
NF = F // FT

NC = 2
NS = 16
NW = NC * NS

_INV_SQRT2 = 0.7071067811865476


def _gating_body(x_ref, wg_ref, out_ref):
    x = x_ref[...]
    wg = wg_ref[...]
    logits = lax.dot_general(x, wg, (((1,), (1,)), ((), ())),
                             preferred_element_type=jnp.float32)
    col = lax.broadcasted_iota(jnp.int32, logits.shape, 1)
    logits = jnp.where(col < E, logits, jnp.float32(-1e30))
    m = jnp.max(logits, axis=1, keepdims=True)
    p = jnp.exp(logits - m)
    p = p / jnp.sum(p, axis=1, keepdims=True)
    m1 = jnp.max(p, axis=1, keepdims=True)
    i1 = jnp.min(jnp.where(p == m1, col, E), axis=1, keepdims=True)
    pm = jnp.where(col == i1, jnp.float32(-1.0), p)
    m2 = jnp.max(pm, axis=1, keepdims=True)
    i2 = jnp.min(jnp.where(pm == m2, col, E), axis=1, keepdims=True)
    out = jnp.where(col == 0, m1, 0.0)
    out = jnp.where(col == 1, m2, out)
    out = jnp.where(col == 2, i1.astype(jnp.float32), out)
    out = jnp.where(col == 3, i2.astype(jnp.float32), out)
    out_ref[...] = out


def _gating_call(x2d, wg_p, interpret=False):
    return pl.pallas_call(
        _gating_body,
        out_shape=jax.ShapeDtypeStruct((S, 128), jnp.float32),
        interpret=interpret,
    )(x2d, wg_p)


_RPW = P // NW
_CH = 32


def _dispatch_body(x_hbm, tok_hbm, out_hbm, idx_v, rows_v, sem):
    wid = lax.axis_index("s") * NC + lax.axis_index("c")
    base = wid * _RPW

    def chunk(i, carry):
        off = base + i * _CH
        pltpu.sync_copy(tok_hbm.at[pl.ds(off, _CH)], idx_v)
        pltpu.async_copy(x_hbm.at[idx_v], rows_v, sem).wait()
        pltpu.sync_copy(rows_v, out_hbm.at[pl.ds(off, _CH)])
        return carry

    lax.fori_loop(0, _RPW // _CH, chunk, 0)


def _dispatch_call(x2d, tok_rows):
    mesh = plsc.VectorSubcoreMesh(core_axis_name="c", subcore_axis_name="s")
    f = functools.partial(
        pl.kernel,
        out_type=jax.ShapeDtypeStruct((P, D), jnp.float32),
        mesh=mesh,
        scratch_types=[
            pltpu.VMEM((_CH,), jnp.int32),
            pltpu.VMEM((_CH, D), jnp.float32),
            pltpu.SemaphoreType.DMA,
        ],
    )(_dispatch_body)
    return f(x2d, tok_rows)


def _ffn_body(te_ref, tv_ref, xs_ref, w1_ref, b1_ref, w2_ref, b2_ref,
              wr_ref, out_ref, xb_ref, w1b_ref, w2b_ref, hb_ref):
    f = pl.program_id(0)
    t = pl.program_id(1)

    @pl.when(tv_ref[t] == 1)
    def _():
        r0 = pl.multiple_of(t * T, T)

        @pl.when(f == 0)
        def _():
            xb_ref[pl.ds(r0, T), :] = xs_ref[...].astype(jnp.bfloat16)

        new_w = jnp.logical_or(
            t == 0, te_ref[t] != te_ref[jnp.maximum(t - 1, 0)])

        @pl.when(new_w)
        def _():
            w1b_ref[...] = w1_ref[0].astype(jnp.bfloat16)
            w2b_ref[...] = w2_ref[0].astype(jnp.bfloat16)

        x = xb_ref[pl.ds(r0, T), :]
        h = jnp.dot(x, w1b_ref[...], preferred_element_type=jnp.float32)
        h = h + b1_ref[0]
        h = 0.5 * h * (1.0 + lax.erf(h * _INV_SQRT2))
        hb_ref[...] = h.astype(jnp.bfloat16)
        o = jnp.dot(hb_ref[...], w2b_ref[...],
                    preferred_element_type=jnp.float32)
        w = wr_ref[...]

        @pl.when(f == 0)
        def _():
            out_ref[pl.ds(r0, T), :] = (o + b2_ref[0]) * w

        @pl.when(f != 0)
        def _():
            out_ref[pl.ds(r0, T), :] = out_ref[pl.ds(r0, T), :] + o * w


def _ffn_call(te, tv, xs, W1, b1, W2, b2, w_rows2d, interpret=False):
    grid_spec = pltpu.PrefetchScalarGridSpec(
        num_scalar_prefetch=2,
        grid=(NF, NT),
        in_specs=[
            pl.BlockSpec((T, D), lambda f, t, te, tv: (t, 0)),
            pl.BlockSpec((1, D, FT), lambda f, t, te, tv: (te[t], 0, f)),
            pl.BlockSpec((1, 1, FT), lambda f, t, te, tv: (te[t], 0, f)),
            pl.BlockSpec((1, FT, D), lambda f, t, te, tv: (te[t], f, 0)),
            pl.BlockSpec((1, 1, D), lambda f, t, te, tv: (te[t], 0, 0)),
            pl.BlockSpec((T, 1), lambda f, t, te, tv: (t, 0)),
        ],
        out_specs=pl.BlockSpec((P, D), lambda f, t, te, tv: (0, 0)),
        scratch_shapes=[
            pltpu.VMEM((P, D), jnp.bfloat16),
            pltpu.VMEM((D, FT), jnp.bfloat16),
            pltpu.VMEM((FT, D), jnp.bfloat16),
            pltpu.VMEM((T, FT), jnp.bfloat16),
        ],
    )
    return pl.pallas_call(
        _ffn_body,
        grid_spec=grid_spec,
        out_shape=jax.ShapeDtypeStruct((P, D), jnp.float32),
        compiler_params=pltpu.CompilerParams(
            dimension_semantics=("arbitrary", "arbitrary")),
        interpret=interpret,
    )(te, tv, xs, W1, b1.reshape(E, 1, F), W2, b2.reshape(E, 1, D), w_rows2d)


_TPW = S // NW
_CH2 = 32
_NV = D // 16


def _combine_body(xs_hbm, pa_hbm, pb_hbm, out_hbm, ia_v, ib_v, a_v, b_v, sem):
    wid = lax.axis_index("s") * NC + lax.axis_index("c")
    base = wid * _TPW

    def chunk(i, carry):
        off = base + i * _CH2
        pltpu.sync_copy(pa_hbm.at[pl.ds(off, _CH2)], ia_v)
        pltpu.sync_copy(pb_hbm.at[pl.ds(off, _CH2)], ib_v)
        pltpu.async_copy(xs_hbm.at[ia_v], a_v, sem).wait()
        pltpu.async_copy(xs_hbm.at[ib_v], b_v, sem).wait()

        def vadd(j, c2):
            r = j // _NV
            v = (j % _NV) * 16
            a_v[r, pl.ds(v, 16)] = a_v[r, pl.ds(v, 16)] + b_v[r, pl.ds(v, 16)]
            return c2

        lax.fori_loop(0, _CH2 * _NV, vadd, 0)
        pltpu.sync_copy(a_v, out_hbm.at[pl.ds(off, _CH2)])
        return carry

    lax.fori_loop(0, _TPW // _CH2, chunk, 0)


def _combine_call(xs_out, pos_a, pos_b):
    mesh = plsc.VectorSubcoreMesh(core_axis_name="c", subcore_axis_name="s")
    f = functools.partial(
        pl.kernel,
        out_type=jax.ShapeDtypeStruct((S, D), jnp.float32),
        mesh=mesh,
        scratch_types=[
            pltpu.VMEM((_CH2,), jnp.int32),
            pltpu.VMEM((_CH2,), jnp.int32),
            pltpu.VMEM((_CH2, D), jnp.float32),
            pltpu.VMEM((_CH2, D), jnp.float32),
            pltpu.SemaphoreType.DMA,
        ],
    )(_combine_body)
    return f(xs_out, pos_a, pos_b)


def _routing_meta(gate_out):
    w1t = gate_out[:, 0]
    w2t = gate_out[:, 1]
    e1 = gate_out[:, 2].astype(jnp.int32)
    e2 = gate_out[:, 3].astype(jnp.int32)
    eids = jnp.stack([e1, e2], axis=1).reshape(-1)
    wts = jnp.stack([w1t, w2t], axis=1).reshape(-1)
    oh = (eids[:, None] == jnp.arange(E, dtype=jnp.int32)[None, :])
    ohi = oh.astype(jnp.int32)
    counts = jnp.sum(ohi, axis=0)
    rank = jnp.sum(jnp.cumsum(ohi, axis=0) * ohi, axis=1) - 1
    padded = ((counts + T - 1) // T) * T
    offs = jnp.concatenate(
        [jnp.zeros((1,), jnp.int32), jnp.cumsum(padded)[:-1]])
    pos = offs[eids] + rank
    tok = jnp.arange(2 * S, dtype=jnp.int32) // 2
    tok_rows = jnp.zeros((P,), jnp.int32).at[pos].set(tok)
    w_rows = jnp.zeros((P,), jnp.float32).at[pos].set(wts)
    ntiles = padded // T
    tcum = jnp.cumsum(ntiles)
    tidx = jnp.arange(NT, dtype=jnp.int32)
    te = jnp.searchsorted(tcum, tidx, side="right").astype(jnp.int32)
    te = jnp.minimum(te, E - 1)
    tv = (tidx < tcum[-1]).astype(jnp.int32)
    return tok_rows, w_rows, te, tv, pos


def kernel(x, w_gate, W1, b1, W2, b2):
    x2d = x.reshape(S, D)
    wg_p = jnp.zeros((128, D), jnp.float32).at[:E].set(w_gate)
    gate_out = _gating_call(x2d, wg_p)
    tok_rows, w_rows, te, tv, pos = _routing_meta(gate_out)
    xs = _dispatch_call(x2d, tok_rows)
    xs_out = _ffn_call(te, tv, xs, W1, b1, W2, b2,
                       w_rows.reshape(P, 1))
    pos2 = pos.reshape(S, 2)
    out2d = _combine_call(xs_out, pos2[:, 0], pos2[:, 1])
    return out2d.reshape(1, S, D)


# speedup vs baseline: 1.0709x; 1.0709x over previous
import functools

import jax
import jax.numpy as jnp
from jax import lax
from jax.experimental import pallas as pl
from jax.experimental.pallas import tpu as pltpu
from jax.experimental.pallas import tpu_sc as plsc

D = 1024
E = 8
F = 4096
S = 2048
T = 128
NT = (2 * S + E * (T - 1) + T - 1) // T
P = NT * T
FT = 512
NF = F // FT

NC = 2
NS = 16
NW = NC * NS

_INV_SQRT2 = 0.7071067811865476


def _gating_body(x_ref, wg_ref, out_ref):
    x = x_ref[...]
    wg = wg_ref[...]
    logits = lax.dot_general(x, wg, (((1,), (1,)), ((), ())),
                             preferred_element_type=jnp.float32)
    col = lax.broadcasted_iota(jnp.int32, logits.shape, 1)
    logits = jnp.where(col < E, logits, jnp.float32(-1e30))
    m = jnp.max(logits, axis=1, keepdims=True)
    p = jnp.exp(logits - m)
    p = p / jnp.sum(p, axis=1, keepdims=True)
    m1 = jnp.max(p, axis=1, keepdims=True)
    i1 = jnp.min(jnp.where(p == m1, col, E), axis=1, keepdims=True)
    pm = jnp.where(col == i1, jnp.float32(-1.0), p)
    m2 = jnp.max(pm, axis=1, keepdims=True)
    i2 = jnp.min(jnp.where(pm == m2, col, E), axis=1, keepdims=True)
    out = jnp.where(col == 0, m1, 0.0)
    out = jnp.where(col == 1, m2, out)
    out = jnp.where(col == 2, i1.astype(jnp.float32), out)
    out = jnp.where(col == 3, i2.astype(jnp.float32), out)
    out_ref[...] = out


def _gating_call(x2d, wg_p, interpret=False):
    return pl.pallas_call(
        _gating_body,
        out_shape=jax.ShapeDtypeStruct((S, 128), jnp.float32),
        interpret=interpret,
    )(x2d, wg_p)


_RPW = P // NW
_CH = 40
_NCH = _RPW // _CH


def _dispatch_body(x_hbm, tok_hbm, out_hbm, i0, i1, b0, b1, s0, s1):
    wid = lax.axis_index("s") * NC + lax.axis_index("c")
    base = wid * _RPW
    ibufs = (i0, i1)
    bufs = (b0, b1)
    sems = (s0, s1)

    def start(i):
        pltpu.sync_copy(tok_hbm.at[wid * _NCH + i], ibufs[i % 2])
        return pltpu.async_copy(
            x_hbm.at[ibufs[i % 2]], bufs[i % 2], sems[i % 2])

    h = [start(0), None]
    for i in range(_NCH):
        if i + 1 < _NCH:
            h[(i + 1) % 2] = start(i + 1)
        h[i % 2].wait()
        pltpu.sync_copy(bufs[i % 2], out_hbm.at[pl.ds(base + i * _CH, _CH)])


def _dispatch_call(x2d, tok_rows):
    mesh = plsc.VectorSubcoreMesh(core_axis_name="c", subcore_axis_name="s")
    f = functools.partial(
        pl.kernel,
        out_type=jax.ShapeDtypeStruct((P, D), jnp.float32),
        mesh=mesh,
        scratch_types=[
            pltpu.VMEM((_CH,), jnp.int32),
            pltpu.VMEM((_CH,), jnp.int32),
            pltpu.VMEM((_CH, D), jnp.float32),
            pltpu.VMEM((_CH, D), jnp.float32),
            pltpu.SemaphoreType.DMA,
            pltpu.SemaphoreType.DMA,
        ],
    )(_dispatch_body)
    return f(x2d, tok_rows.reshape(NW * _NCH, _CH))


def _ffn_body(te_ref, tv_ref, xs_ref, w1_ref, b1_ref, w2_ref, b2_ref,
              wr_ref, out_ref):
    f = pl.program_id(0)
    t = pl.program_id(1)

    @pl.when(tv_ref[t] == 1)
    def _():
        r0 = pl.multiple_of(t * T, T)
        x = xs_ref[pl.ds(r0, T), :]
        h = jnp.dot(x, w1_ref[0], preferred_element_type=jnp.float32)
        h = h + b1_ref[0]
        h = 0.5 * h * (1.0 + lax.erf(h * _INV_SQRT2))
        o = jnp.dot(h, w2_ref[0], preferred_element_type=jnp.float32)
        w = wr_ref[...]

        @pl.when(f == 0)
        def _():
            out_ref[pl.ds(r0, T), :] = (o + b2_ref[0]) * w

        @pl.when(f != 0)
        def _():
            out_ref[pl.ds(r0, T), :] = out_ref[pl.ds(r0, T), :] + o * w


def _ffn_call(te, tv, xs, W1, b1, W2, b2, w_rows2d, interpret=False):
    grid_spec = pltpu.PrefetchScalarGridSpec(
        num_scalar_prefetch=2,
        grid=(NF, NT),
        in_specs=[
            pl.BlockSpec((P, D), lambda f, t, te, tv: (0, 0)),
            pl.BlockSpec((1, D, FT), lambda f, t, te, tv: (te[t], 0, f)),
            pl.BlockSpec((1, 1, FT), lambda f, t, te, tv: (te[t], 0, f)),
            pl.BlockSpec((1, FT, D), lambda f, t, te, tv: (te[t], f, 0)),
            pl.BlockSpec((1, 1, D), lambda f, t, te, tv: (te[t], 0, 0)),
            pl.BlockSpec((T, 1), lambda f, t, te, tv: (t, 0)),
        ],
        out_specs=pl.BlockSpec((P, D), lambda f, t, te, tv: (0, 0)),
    )
    return pl.pallas_call(
        _ffn_body,
        grid_spec=grid_spec,
        out_shape=jax.ShapeDtypeStruct((P, D), jnp.float32),
        compiler_params=pltpu.CompilerParams(
            dimension_semantics=("arbitrary", "arbitrary")),
        interpret=interpret,
    )(te, tv, xs, W1, b1.reshape(E, 1, F), W2, b2.reshape(E, 1, D), w_rows2d)


_TPW = S // NW
_CH2 = 16
_NCH2 = _TPW // _CH2
_NV = D // 16


def _combine_body(xs_hbm, pa_hbm, pb_hbm, out_hbm,
                  ia0, ia1, ib0, ib1, a0, a1, b0, b1, sa0, sa1, sb0, sb1):
    wid = lax.axis_index("s") * NC + lax.axis_index("c")
    base = wid * _TPW
    iabufs, ibbufs = (ia0, ia1), (ib0, ib1)
    abufs, asems = (a0, a1), (sa0, sa1)
    bbufs, bsems = (b0, b1), (sb0, sb1)

    def start_a(i):
        pltpu.sync_copy(pa_hbm.at[wid * _NCH2 + i], iabufs[i % 2])
        return pltpu.async_copy(xs_hbm.at[iabufs[i % 2]], abufs[i % 2],
                                asems[i % 2])

    def start_b(i):
        pltpu.sync_copy(pb_hbm.at[wid * _NCH2 + i], ibbufs[i % 2])
        return pltpu.async_copy(xs_hbm.at[ibbufs[i % 2]], bbufs[i % 2],
                                bsems[i % 2])

    ha = [start_a(0), None]
    hb = [start_b(0), None]
    for i in range(_NCH2):
        if i + 1 < _NCH2:
            ha[(i + 1) % 2] = start_a(i + 1)
            hb[(i + 1) % 2] = start_b(i + 1)
        ha[i % 2].wait()
        hb[i % 2].wait()
        a_v, b_v = abufs[i % 2], bbufs[i % 2]

        def vadd(j, c2):
            r = j // _NV
            v = (j % _NV) * 16
            a_v[r, pl.ds(v, 16)] = a_v[r, pl.ds(v, 16)] + b_v[r, pl.ds(v, 16)]
            return c2

        lax.fori_loop(0, _CH2 * _NV, vadd, 0)
        pltpu.sync_copy(a_v, out_hbm.at[pl.ds(base + i * _CH2, _CH2)])


def _combine_call(xs_out, pos_a, pos_b):
    mesh = plsc.VectorSubcoreMesh(core_axis_name="c", subcore_axis_name="s")
    f = functools.partial(
        pl.kernel,
        out_type=jax.ShapeDtypeStruct((S, D), jnp.float32),
        mesh=mesh,
        scratch_types=[
            pltpu.VMEM((_CH2,), jnp.int32),
            pltpu.VMEM((_CH2,), jnp.int32),
            pltpu.VMEM((_CH2,), jnp.int32),
            pltpu.VMEM((_CH2,), jnp.int32),
            pltpu.VMEM((_CH2, D), jnp.float32),
            pltpu.VMEM((_CH2, D), jnp.float32),
            pltpu.VMEM((_CH2, D), jnp.float32),
            pltpu.VMEM((_CH2, D), jnp.float32),
            pltpu.SemaphoreType.DMA,
            pltpu.SemaphoreType.DMA,
            pltpu.SemaphoreType.DMA,
            pltpu.SemaphoreType.DMA,
        ],
    )(_combine_body)
    return f(xs_out, pos_a.reshape(NW * _NCH2, _CH2),
             pos_b.reshape(NW * _NCH2, _CH2))


def _routing_meta(gate_out):
    w1t = gate_out[:, 0]
    w2t = gate_out[:, 1]
    e1 = gate_out[:, 2].astype(jnp.int32)
    e2 = gate_out[:, 3].astype(jnp.int32)
    eids = jnp.stack([e1, e2], axis=1).reshape(-1)
    wts = jnp.stack([w1t, w2t], axis=1).reshape(-1)
    oh = (eids[:, None] == jnp.arange(E, dtype=jnp.int32)[None, :])
    ohi = oh.astype(jnp.int32)
    counts = jnp.sum(ohi, axis=0)
    rank = jnp.sum(jnp.cumsum(ohi, axis=0) * ohi, axis=1) - 1
    padded = ((counts + T - 1) // T) * T
    offs = jnp.concatenate(
        [jnp.zeros((1,), jnp.int32), jnp.cumsum(padded)[:-1]])
    pos = offs[eids] + rank
    tok = jnp.arange(2 * S, dtype=jnp.int32) // 2
    tok_rows = jnp.zeros((P,), jnp.int32).at[pos].set(tok)
    w_rows = jnp.zeros((P,), jnp.float32).at[pos].set(wts)
    ntiles = padded // T
    tcum = jnp.cumsum(ntiles)
    tidx = jnp.arange(NT, dtype=jnp.int32)
    te = jnp.searchsorted(tcum, tidx, side="right").astype(jnp.int32)
    te = jnp.minimum(te, E - 1)
    tv = (tidx < tcum[-1]).astype(jnp.int32)
    return tok_rows, w_rows, te, tv, pos


def kernel(x, w_gate, W1, b1, W2, b2):
    x2d = x.reshape(S, D)
    wg_p = jnp.zeros((128, D), jnp.float32).at[:E].set(w_gate)
    gate_out = _gating_call(x2d, wg_p)
    tok_rows, w_rows, te, tv, pos = _routing_meta(gate_out)
    xs = _dispatch_call(x2d, tok_rows)
    xs_out = _ffn_call(te, tv, xs, W1, b1, W2, b2,
                       w_rows.reshape(P, 1))
    pos2 = pos.reshape(S, 2)
    out2d = _combine_call(xs_out, pos2[:, 0], pos2[:, 1])
    return out2d.reshape(1, S, D)


# speedup vs baseline: 1.3820x; 1.2905x over previous
import functools

import jax
import jax.numpy as jnp
from jax import lax
from jax.experimental import pallas as pl
from jax.experimental.pallas import tpu as pltpu
from jax.experimental.pallas import tpu_sc as plsc

D = 1024
E = 8
F = 4096
S = 2048
T = 128
NT = (2 * S + E * (T - 1) + T - 1) // T
P = NT * T
FT = 1024
NF = F // FT

NC = 2
NS = 16
NW = NC * NS

_INV_SQRT2 = 0.7071067811865476


def _gating_body(x_ref, wg_ref, out_ref):
    x = x_ref[...]
    wg = wg_ref[...]
    logits = lax.dot_general(x, wg, (((1,), (1,)), ((), ())),
                             preferred_element_type=jnp.float32)
    col = lax.broadcasted_iota(jnp.int32, logits.shape, 1)
    logits = jnp.where(col < E, logits, jnp.float32(-1e30))
    m = jnp.max(logits, axis=1, keepdims=True)
    p = jnp.exp(logits - m)
    p = p / jnp.sum(p, axis=1, keepdims=True)
    m1 = jnp.max(p, axis=1, keepdims=True)
    i1 = jnp.min(jnp.where(p == m1, col, E), axis=1, keepdims=True)
    pm = jnp.where(col == i1, jnp.float32(-1.0), p)
    m2 = jnp.max(pm, axis=1, keepdims=True)
    i2 = jnp.min(jnp.where(pm == m2, col, E), axis=1, keepdims=True)
    out = jnp.where(col == 0, m1, 0.0)
    out = jnp.where(col == 1, m2, out)
    out = jnp.where(col == 2, i1.astype(jnp.float32), out)
    out = jnp.where(col == 3, i2.astype(jnp.float32), out)
    out_ref[...] = out


def _gating_call(x2d, wg_p, interpret=False):
    return pl.pallas_call(
        _gating_body,
        out_shape=jax.ShapeDtypeStruct((S, 128), jnp.float32),
        interpret=interpret,
    )(x2d, wg_p)


_RPW = P // NW
_CH = 40
_NCH = _RPW // _CH


def _dispatch_body(x_hbm, tok_hbm, out_hbm, i0, i1, b0, b1, s0, s1):
    wid = lax.axis_index("s") * NC + lax.axis_index("c")
    base = wid * _RPW
    ibufs = (i0, i1)
    bufs = (b0, b1)
    sems = (s0, s1)

    def start(i):
        pltpu.sync_copy(tok_hbm.at[wid * _NCH + i], ibufs[i % 2])
        return pltpu.async_copy(
            x_hbm.at[ibufs[i % 2]], bufs[i % 2], sems[i % 2])

    h = [start(0), None]
    for i in range(_NCH):
        if i + 1 < _NCH:
            h[(i + 1) % 2] = start(i + 1)
        h[i % 2].wait()
        pltpu.sync_copy(bufs[i % 2], out_hbm.at[pl.ds(base + i * _CH, _CH)])


def _dispatch_call(x2d, tok_rows):
    mesh = plsc.VectorSubcoreMesh(core_axis_name="c", subcore_axis_name="s")
    f = functools.partial(
        pl.kernel,
        out_type=jax.ShapeDtypeStruct((P, D), jnp.float32),
        mesh=mesh,
        scratch_types=[
            pltpu.VMEM((_CH,), jnp.int32),
            pltpu.VMEM((_CH,), jnp.int32),
            pltpu.VMEM((_CH, D), jnp.float32),
            pltpu.VMEM((_CH, D), jnp.float32),
            pltpu.SemaphoreType.DMA,
            pltpu.SemaphoreType.DMA,
        ],
    )(_dispatch_body)
    return f(x2d, tok_rows.reshape(NW * _NCH, _CH))


def _ffn_body(te_ref, tv_ref, xs_ref, w1_ref, b1_ref, w2_ref, b2_ref,
              wr_ref, out_ref):
    f = pl.program_id(0)
    t = pl.program_id(1)

    @pl.when(tv_ref[t] == 1)
    def _():
        r0 = pl.multiple_of(t * T, T)
        x = xs_ref[pl.ds(r0, T), :]
        h = jnp.dot(x, w1_ref[0], preferred_element_type=jnp.float32)
        h = h + b1_ref[0]
        h = 0.5 * h * (1.0 + lax.erf(h * _INV_SQRT2))
        o = jnp.dot(h, w2_ref[0], preferred_element_type=jnp.float32)
        w = wr_ref[...]

        @pl.when(f == 0)
        def _():
            out_ref[pl.ds(r0, T), :] = (o + b2_ref[0]) * w

        @pl.when(f != 0)
        def _():
            out_ref[pl.ds(r0, T), :] = out_ref[pl.ds(r0, T), :] + o * w


def _ffn_call(te, tv, xs, W1, b1, W2, b2, w_rows2d, interpret=False):
    grid_spec = pltpu.PrefetchScalarGridSpec(
        num_scalar_prefetch=2,
        grid=(NF, NT),
        in_specs=[
            pl.BlockSpec((P, D), lambda f, t, te, tv: (0, 0)),
            pl.BlockSpec((1, D, FT), lambda f, t, te, tv: (te[t], 0, f)),
            pl.BlockSpec((1, 1, FT), lambda f, t, te, tv: (te[t], 0, f)),
            pl.BlockSpec((1, FT, D), lambda f, t, te, tv: (te[t], f, 0)),
            pl.BlockSpec((1, 1, D), lambda f, t, te, tv: (te[t], 0, 0)),
            pl.BlockSpec((T, 1), lambda f, t, te, tv: (t, 0)),
        ],
        out_specs=pl.BlockSpec((P, D), lambda f, t, te, tv: (0, 0)),
    )
    return pl.pallas_call(
        _ffn_body,
        grid_spec=grid_spec,
        out_shape=jax.ShapeDtypeStruct((P, D), jnp.float32),
        compiler_params=pltpu.CompilerParams(
            dimension_semantics=("arbitrary", "arbitrary")),
        interpret=interpret,
    )(te, tv, xs, W1, b1.reshape(E, 1, F), W2, b2.reshape(E, 1, D), w_rows2d)


_TPW = S // NW
_CH2 = 16
_NCH2 = _TPW // _CH2
_NV = D // 16


def _combine_body(xs_hbm, pa_hbm, pb_hbm, out_hbm,
                  ia0, ia1, ib0, ib1, a0, a1, b0, b1, sa0, sa1, sb0, sb1):
    wid = lax.axis_index("s") * NC + lax.axis_index("c")
    base = wid * _TPW
    iabufs, ibbufs = (ia0, ia1), (ib0, ib1)
    abufs, asems = (a0, a1), (sa0, sa1)
    bbufs, bsems = (b0, b1), (sb0, sb1)

    def start_a(i):
        pltpu.sync_copy(pa_hbm.at[wid * _NCH2 + i], iabufs[i % 2])
        return pltpu.async_copy(xs_hbm.at[iabufs[i % 2]], abufs[i % 2],
                                asems[i % 2])

    def start_b(i):
        pltpu.sync_copy(pb_hbm.at[wid * _NCH2 + i], ibbufs[i % 2])
        return pltpu.async_copy(xs_hbm.at[ibbufs[i % 2]], bbufs[i % 2],
                                bsems[i % 2])

    ha = [start_a(0), None]
    hb = [start_b(0), None]
    for i in range(_NCH2):
        if i + 1 < _NCH2:
            ha[(i + 1) % 2] = start_a(i + 1)
            hb[(i + 1) % 2] = start_b(i + 1)
        ha[i % 2].wait()
        hb[i % 2].wait()
        a_v, b_v = abufs[i % 2], bbufs[i % 2]

        def vadd(j, c2):
            r = j // _NV
            v = (j % _NV) * 16
            a_v[r, pl.ds(v, 16)] = a_v[r, pl.ds(v, 16)] + b_v[r, pl.ds(v, 16)]
            return c2

        lax.fori_loop(0, _CH2 * _NV, vadd, 0)
        pltpu.sync_copy(a_v, out_hbm.at[pl.ds(base + i * _CH2, _CH2)])


def _combine_call(xs_out, pos_a, pos_b):
    mesh = plsc.VectorSubcoreMesh(core_axis_name="c", subcore_axis_name="s")
    f = functools.partial(
        pl.kernel,
        out_type=jax.ShapeDtypeStruct((S, D), jnp.float32),
        mesh=mesh,
        scratch_types=[
            pltpu.VMEM((_CH2,), jnp.int32),
            pltpu.VMEM((_CH2,), jnp.int32),
            pltpu.VMEM((_CH2,), jnp.int32),
            pltpu.VMEM((_CH2,), jnp.int32),
            pltpu.VMEM((_CH2, D), jnp.float32),
            pltpu.VMEM((_CH2, D), jnp.float32),
            pltpu.VMEM((_CH2, D), jnp.float32),
            pltpu.VMEM((_CH2, D), jnp.float32),
            pltpu.SemaphoreType.DMA,
            pltpu.SemaphoreType.DMA,
            pltpu.SemaphoreType.DMA,
            pltpu.SemaphoreType.DMA,
        ],
    )(_combine_body)
    return f(xs_out, pos_a.reshape(NW * _NCH2, _CH2),
             pos_b.reshape(NW * _NCH2, _CH2))


def _routing_meta(gate_out):
    w1t = gate_out[:, 0]
    w2t = gate_out[:, 1]
    e1 = gate_out[:, 2].astype(jnp.int32)
    e2 = gate_out[:, 3].astype(jnp.int32)
    eids = jnp.stack([e1, e2], axis=1).reshape(-1)
    wts = jnp.stack([w1t, w2t], axis=1).reshape(-1)
    oh = (eids[:, None] == jnp.arange(E, dtype=jnp.int32)[None, :])
    ohi = oh.astype(jnp.int32)
    counts = jnp.sum(ohi, axis=0)
    rank = jnp.sum(jnp.cumsum(ohi, axis=0) * ohi, axis=1) - 1
    padded = ((counts + T - 1) // T) * T
    offs = jnp.concatenate(
        [jnp.zeros((1,), jnp.int32), jnp.cumsum(padded)[:-1]])
    pos = offs[eids] + rank
    tok = jnp.arange(2 * S, dtype=jnp.int32) // 2
    tok_rows = jnp.zeros((P,), jnp.int32).at[pos].set(tok)
    w_rows = jnp.zeros((P,), jnp.float32).at[pos].set(wts)
    ntiles = padded // T
    tcum = jnp.cumsum(ntiles)
    tidx = jnp.arange(NT, dtype=jnp.int32)
    te = jnp.searchsorted(tcum, tidx, side="right").astype(jnp.int32)
    te = jnp.minimum(te, E - 1)
    tv = (tidx < tcum[-1]).astype(jnp.int32)
    return tok_rows, w_rows, te, tv, pos


def kernel(x, w_gate, W1, b1, W2, b2):
    x2d = x.reshape(S, D)
    wg_p = jnp.zeros((128, D), jnp.float32).at[:E].set(w_gate)
    gate_out = _gating_call(x2d, wg_p)
    tok_rows, w_rows, te, tv, pos = _routing_meta(gate_out)
    xs = _dispatch_call(x2d, tok_rows)
    xs_out = _ffn_call(te, tv, xs, W1, b1, W2, b2,
                       w_rows.reshape(P, 1))
    pos2 = pos.reshape(S, 2)
    out2d = _combine_call(xs_out, pos2[:, 0], pos2[:, 1])
    return out2d.reshape(1, S, D)


# speedup vs baseline: 1.5931x; 1.1528x over previous
import functools

import jax
import jax.numpy as jnp
from jax import lax
from jax.experimental import pallas as pl
from jax.experimental.pallas import tpu as pltpu
from jax.experimental.pallas import tpu_sc as plsc

D = 1024
E = 8
F = 4096
S = 2048
T = 128
NT = (2 * S + E * (T - 1) + T - 1) // T
P = NT * T
FT = 2048
NF = F // FT

NC = 2
NS = 16
NW = NC * NS

_INV_SQRT2 = 0.7071067811865476


def _gating_body(x_ref, wg_ref, out_ref):
    x = x_ref[...]
    wg = wg_ref[...]
    logits = lax.dot_general(x, wg, (((1,), (1,)), ((), ())),
                             preferred_element_type=jnp.float32)
    col = lax.broadcasted_iota(jnp.int32, logits.shape, 1)
    logits = jnp.where(col < E, logits, jnp.float32(-1e30))
    m = jnp.max(logits, axis=1, keepdims=True)
    p = jnp.exp(logits - m)
    p = p / jnp.sum(p, axis=1, keepdims=True)
    m1 = jnp.max(p, axis=1, keepdims=True)
    i1 = jnp.min(jnp.where(p == m1, col, E), axis=1, keepdims=True)
    pm = jnp.where(col == i1, jnp.float32(-1.0), p)
    m2 = jnp.max(pm, axis=1, keepdims=True)
    i2 = jnp.min(jnp.where(pm == m2, col, E), axis=1, keepdims=True)
    out = jnp.where(col == 0, m1, 0.0)
    out = jnp.where(col == 1, m2, out)
    out = jnp.where(col == 2, i1.astype(jnp.float32), out)
    out = jnp.where(col == 3, i2.astype(jnp.float32), out)
    out_ref[...] = out


def _gating_call(x2d, wg_p, interpret=False):
    return pl.pallas_call(
        _gating_body,
        out_shape=jax.ShapeDtypeStruct((S, 128), jnp.float32),
        interpret=interpret,
    )(x2d, wg_p)


_RPW = P // NW
_CH = 40
_NCH = _RPW // _CH


def _dispatch_body(x_hbm, tok_hbm, out_hbm, i0, i1, b0, b1, s0, s1):
    wid = lax.axis_index("s") * NC + lax.axis_index("c")
    base = wid * _RPW
    ibufs = (i0, i1)
    bufs = (b0, b1)
    sems = (s0, s1)

    def start(i):
        pltpu.sync_copy(tok_hbm.at[wid * _NCH + i], ibufs[i % 2])
        return pltpu.async_copy(
            x_hbm.at[ibufs[i % 2]], bufs[i % 2], sems[i % 2])

    h = [start(0), None]
    for i in range(_NCH):
        if i + 1 < _NCH:
            h[(i + 1) % 2] = start(i + 1)
        h[i % 2].wait()
        pltpu.sync_copy(bufs[i % 2], out_hbm.at[pl.ds(base + i * _CH, _CH)])


def _dispatch_call(x2d, tok_rows):
    mesh = plsc.VectorSubcoreMesh(core_axis_name="c", subcore_axis_name="s")
    f = functools.partial(
        pl.kernel,
        out_type=jax.ShapeDtypeStruct((P, D), jnp.float32),
        mesh=mesh,
        scratch_types=[
            pltpu.VMEM((_CH,), jnp.int32),
            pltpu.VMEM((_CH,), jnp.int32),
            pltpu.VMEM((_CH, D), jnp.float32),
            pltpu.VMEM((_CH, D), jnp.float32),
            pltpu.SemaphoreType.DMA,
            pltpu.SemaphoreType.DMA,
        ],
    )(_dispatch_body)
    return f(x2d, tok_rows.reshape(NW * _NCH, _CH))


def _ffn_body(te_ref, tv_ref, xs_ref, w1_ref, b1_ref, w2_ref, b2_ref,
              wr_ref, out_ref):
    f = pl.program_id(0)
    t = pl.program_id(1)

    @pl.when(tv_ref[t] == 1)
    def _():
        r0 = pl.multiple_of(t * T, T)
        x = xs_ref[...]
        h = jnp.dot(x, w1_ref[0], preferred_element_type=jnp.float32)
        h = h + b1_ref[0]
        h = 0.5 * h * (1.0 + lax.erf(h * _INV_SQRT2))
        o = jnp.dot(h, w2_ref[0], preferred_element_type=jnp.float32)
        w = wr_ref[...]

        @pl.when(f == 0)
        def _():
            out_ref[pl.ds(r0, T), :] = (o + b2_ref[0]) * w

        @pl.when(f != 0)
        def _():
            out_ref[pl.ds(r0, T), :] = out_ref[pl.ds(r0, T), :] + o * w


def _ffn_call(te, tv, xs, W1, b1, W2, b2, w_rows2d, interpret=False):
    grid_spec = pltpu.PrefetchScalarGridSpec(
        num_scalar_prefetch=2,
        grid=(NF, NT),
        in_specs=[
            pl.BlockSpec((T, D), lambda f, t, te, tv: (t, 0)),
            pl.BlockSpec((1, D, FT), lambda f, t, te, tv: (te[t], 0, f)),
            pl.BlockSpec((1, 1, FT), lambda f, t, te, tv: (te[t], 0, f)),
            pl.BlockSpec((1, FT, D), lambda f, t, te, tv: (te[t], f, 0)),
            pl.BlockSpec((1, 1, D), lambda f, t, te, tv: (te[t], 0, 0)),
            pl.BlockSpec((T, 1), lambda f, t, te, tv: (t, 0)),
        ],
        out_specs=pl.BlockSpec((P, D), lambda f, t, te, tv: (0, 0)),
    )
    return pl.pallas_call(
        _ffn_body,
        grid_spec=grid_spec,
        out_shape=jax.ShapeDtypeStruct((P, D), jnp.float32),
        compiler_params=pltpu.CompilerParams(
            dimension_semantics=("arbitrary", "arbitrary")),
        interpret=interpret,
    )(te, tv, xs, W1, b1.reshape(E, 1, F), W2, b2.reshape(E, 1, D), w_rows2d)


_TPW = S // NW
_CH2 = 16
_NCH2 = _TPW // _CH2
_NV = D // 16


def _combine_body(xs_hbm, pa_hbm, pb_hbm, out_hbm,
                  ia0, ia1, ib0, ib1, a0, a1, b0, b1, sa0, sa1, sb0, sb1):
    wid = lax.axis_index("s") * NC + lax.axis_index("c")
    base = wid * _TPW
    iabufs, ibbufs = (ia0, ia1), (ib0, ib1)
    abufs, asems = (a0, a1), (sa0, sa1)
    bbufs, bsems = (b0, b1), (sb0, sb1)

    def start_a(i):
        pltpu.sync_copy(pa_hbm.at[wid * _NCH2 + i], iabufs[i % 2])
        return pltpu.async_copy(xs_hbm.at[iabufs[i % 2]], abufs[i % 2],
                                asems[i % 2])

    def start_b(i):
        pltpu.sync_copy(pb_hbm.at[wid * _NCH2 + i], ibbufs[i % 2])
        return pltpu.async_copy(xs_hbm.at[ibbufs[i % 2]], bbufs[i % 2],
                                bsems[i % 2])

    ha = [start_a(0), None]
    hb = [start_b(0), None]
    for i in range(_NCH2):
        if i + 1 < _NCH2:
            ha[(i + 1) % 2] = start_a(i + 1)
            hb[(i + 1) % 2] = start_b(i + 1)
        ha[i % 2].wait()
        hb[i % 2].wait()
        a_v, b_v = abufs[i % 2], bbufs[i % 2]

        def vadd(j, c2):
            r = j // _NV
            v = (j % _NV) * 16
            a_v[r, pl.ds(v, 16)] = a_v[r, pl.ds(v, 16)] + b_v[r, pl.ds(v, 16)]
            return c2

        lax.fori_loop(0, _CH2 * _NV, vadd, 0)
        pltpu.sync_copy(a_v, out_hbm.at[pl.ds(base + i * _CH2, _CH2)])


def _combine_call(xs_out, pos_a, pos_b):
    mesh = plsc.VectorSubcoreMesh(core_axis_name="c", subcore_axis_name="s")
    f = functools.partial(
        pl.kernel,
        out_type=jax.ShapeDtypeStruct((S, D), jnp.float32),
        mesh=mesh,
        scratch_types=[
            pltpu.VMEM((_CH2,), jnp.int32),
            pltpu.VMEM((_CH2,), jnp.int32),
            pltpu.VMEM((_CH2,), jnp.int32),
            pltpu.VMEM((_CH2,), jnp.int32),
            pltpu.VMEM((_CH2, D), jnp.float32),
            pltpu.VMEM((_CH2, D), jnp.float32),
            pltpu.VMEM((_CH2, D), jnp.float32),
            pltpu.VMEM((_CH2, D), jnp.float32),
            pltpu.SemaphoreType.DMA,
            pltpu.SemaphoreType.DMA,
            pltpu.SemaphoreType.DMA,
            pltpu.SemaphoreType.DMA,
        ],
    )(_combine_body)
    return f(xs_out, pos_a.reshape(NW * _NCH2, _CH2),
             pos_b.reshape(NW * _NCH2, _CH2))


def _routing_meta(gate_out):
    w1t = gate_out[:, 0]
    w2t = gate_out[:, 1]
    e1 = gate_out[:, 2].astype(jnp.int32)
    e2 = gate_out[:, 3].astype(jnp.int32)
    eids = jnp.stack([e1, e2], axis=1).reshape(-1)
    wts = jnp.stack([w1t, w2t], axis=1).reshape(-1)
    oh = (eids[:, None] == jnp.arange(E, dtype=jnp.int32)[None, :])
    ohi = oh.astype(jnp.int32)
    counts = jnp.sum(ohi, axis=0)
    rank = jnp.sum(jnp.cumsum(ohi, axis=0) * ohi, axis=1) - 1
    padded = ((counts + T - 1) // T) * T
    offs = jnp.concatenate(
        [jnp.zeros((1,), jnp.int32), jnp.cumsum(padded)[:-1]])
    pos = offs[eids] + rank
    tok = jnp.arange(2 * S, dtype=jnp.int32) // 2
    tok_rows = jnp.zeros((P,), jnp.int32).at[pos].set(tok)
    w_rows = jnp.zeros((P,), jnp.float32).at[pos].set(wts)
    ntiles = padded // T
    tcum = jnp.cumsum(ntiles)
    tidx = jnp.arange(NT, dtype=jnp.int32)
    te = jnp.searchsorted(tcum, tidx, side="right").astype(jnp.int32)
    te = jnp.minimum(te, E - 1)
    tv = (tidx < tcum[-1]).astype(jnp.int32)
    return tok_rows, w_rows, te, tv, pos


def kernel(x, w_gate, W1, b1, W2, b2):
    x2d = x.reshape(S, D)
    wg_p = jnp.zeros((128, D), jnp.float32).at[:E].set(w_gate)
    gate_out = _gating_call(x2d, wg_p)
    tok_rows, w_rows, te, tv, pos = _routing_meta(gate_out)
    xs = _dispatch_call(x2d, tok_rows)
    xs_out = _ffn_call(te, tv, xs, W1, b1, W2, b2,
                       w_rows.reshape(P, 1))
    pos2 = pos.reshape(S, 2)
    out2d = _combine_call(xs_out, pos2[:, 0], pos2[:, 1])
    return out2d.reshape(1, S, D)
